# baseline (speedup 1.0000x reference)
import jax
import jax.numpy as jnp
from jax.experimental import pallas as pl
from jax.experimental.pallas import tpu as pltpu

_UNROLL = 16


def _vmse_kernel(nv_ref, a_ref, b_ref, w_ref, o_ref, acc_ref):
    n_pts = a_ref.shape[0]
    bi = a_ref.shape[2]
    b_tot = b_ref.shape[2]

    def dist(n):
        a = a_ref[pl.ds(n, 1)].reshape(16, bi)
        bb = b_ref[pl.ds(n, 1)].reshape(16, b_tot)
        d2 = jax.lax.dot_general(a, bb, (((0,), (0,)), ((), ())),
                                 preferred_element_type=jnp.float32)
        mb = jnp.maximum(d2.astype(jnp.bfloat16), jnp.bfloat16(1e-30))
        return mb * jax.lax.rsqrt(mb)

    def body(k, _):
        n0 = k * _UNROLL
        s = dist(n0)
        for u in range(1, _UNROLL):
            s = s + dist(n0 + u)
        sf = s.astype(jnp.float32)

        @pl.when(k == 0)
        def _init():
            acc_ref[:] = sf

        @pl.when(k > 0)
        def _accum():
            acc_ref[:] = acc_ref[:] + sf

        return 0

    jax.lax.fori_loop(0, n_pts // _UNROLL, body, 0)
    acc = acc_ref[:]

    nv = nv_ref[0, 0]
    norms = acc * (1.0 / n_pts)
    logits = (norms * norms) * (-0.5 / nv)
    m = jnp.max(logits, axis=1, keepdims=True)
    ex = jnp.exp(logits - m)
    lse = jnp.log(jnp.sum(ex, axis=1, keepdims=True)) + m
    rows = jax.lax.broadcasted_iota(jnp.int32, (bi, b_tot), 0)
    cols = jax.lax.broadcasted_iota(jnp.int32, (bi, b_tot), 1)
    diag = jnp.sum(jnp.where(rows == cols, logits, 0.0), axis=1,
                   keepdims=True)
    loss = (lse - diag) * (2.0 * nv) * w_ref[:, :1]
    o_ref[:, :] = jnp.broadcast_to(loss, (bi, 128))


@jax.jit
def kernel(pred, gt, weights, sigma):
    B, N, D = pred.shape
    f32 = jnp.float32
    bf16 = jnp.bfloat16
    pred = pred.astype(f32)
    gt = gt.astype(f32)

    pt = pred.transpose(1, 2, 0)
    gtt = gt.transpose(1, 2, 0)
    ph = pt.astype(bf16)
    pl_ = (pt - ph.astype(f32)).astype(bf16)
    g2t = -2.0 * gtt
    gh = g2t.astype(bf16)
    gl = (g2t - gh.astype(f32)).astype(bf16)

    p2 = jnp.sum(pred * pred, axis=2).T[:, None, :]
    g2 = jnp.sum(gt * gt, axis=2).T[:, None, :]
    p2h = p2.astype(bf16)
    p2l = (p2 - p2h.astype(f32)).astype(bf16)
    g2h = g2.astype(bf16)
    g2l = (g2 - g2h.astype(f32)).astype(bf16)
    ones = jnp.ones((N, 1, B), bf16)
    zeros = jnp.zeros((N, 3, B), bf16)

    a_full = jnp.concatenate(
        [ph, ph, pl_, p2h, p2l, ones, ones, zeros], axis=1)
    b_full = jnp.concatenate(
        [gh, gl, gh, ones, ones, g2h, g2l, zeros], axis=1)

    nv = (sigma.astype(f32) * sigma.astype(f32)).reshape(1, 1)
    w2 = weights.astype(f32).reshape(B, 1)

    out = pl.pallas_call(
        _vmse_kernel,
        grid=(1,),
        in_specs=[
            pl.BlockSpec(memory_space=pltpu.SMEM),
            pl.BlockSpec((N, 16, B), lambda i: (0, 0, 0)),
            pl.BlockSpec((N, 16, B), lambda i: (0, 0, 0)),
            pl.BlockSpec((B, 1), lambda i: (0, 0)),
        ],
        out_specs=pl.BlockSpec((B, 128), lambda i: (0, 0)),
        out_shape=jax.ShapeDtypeStruct((B, 128), f32),
        scratch_shapes=[pltpu.VMEM((B, B), f32)],
        compiler_params=pltpu.CompilerParams(
            dimension_semantics=("arbitrary",),
            vmem_limit_bytes=48 * 1024 * 1024,
        ),
    )(nv, a_full, b_full, w2)
    return out[:, 0]


# speedup vs baseline: 1.8255x; 1.8255x over previous
import jax
import jax.numpy as jnp
from jax.experimental import pallas as pl
from jax.experimental.pallas import tpu as pltpu

_UNROLL = 16


def _vmse_kernel(nv_ref, a_ref, b_ref, w_ref, o_ref, acc_ref):
    n_pts = a_ref.shape[0]
    bi = a_ref.shape[2]
    b_tot = b_ref.shape[2]

    def dist(n):
        a = a_ref[pl.ds(n, 1)].reshape(8, bi)
        bb = b_ref[pl.ds(n, 1)].reshape(8, b_tot)
        d2 = jax.lax.dot_general(a, bb, (((0,), (0,)), ((), ())),
                                 preferred_element_type=jnp.float32)
        mb = jnp.maximum(d2.astype(jnp.bfloat16),
                         jnp.bfloat16(1e-30))
        return mb * jax.lax.rsqrt(mb)

    acc_ref[:] = jnp.zeros((bi, b_tot), jnp.float32)

    def body(k, _):
        n0 = k * _UNROLL
        s = dist(n0)
        for u in range(1, _UNROLL):
            s = s + dist(n0 + u)
        acc_ref[:] = acc_ref[:] + s.astype(jnp.float32)
        return 0

    jax.lax.fori_loop(0, n_pts // _UNROLL, body, 0)
    acc = acc_ref[:]

    nv = nv_ref[0, 0]
    norms = acc * (1.0 / n_pts)
    logits = (norms * norms) * (-0.5 / nv)
    m = jnp.max(logits, axis=1, keepdims=True)
    ex = jnp.exp(logits - m)
    lse = jnp.log(jnp.sum(ex, axis=1, keepdims=True)) + m
    rows = jax.lax.broadcasted_iota(jnp.int32, (bi, b_tot), 0)
    cols = jax.lax.broadcasted_iota(jnp.int32, (bi, b_tot), 1)
    diag = jnp.sum(jnp.where(rows == cols, logits, 0.0), axis=1,
                   keepdims=True)
    loss = (lse - diag) * (2.0 * nv) * w_ref[:, :1]
    o_ref[:, :] = jnp.broadcast_to(loss, (bi, 128))


@jax.jit
def kernel(pred, gt, weights, sigma):
    B, N, D = pred.shape
    f32 = jnp.float32
    pred = pred.astype(f32)
    gt = gt.astype(f32)

    pt = pred.transpose(1, 2, 0)
    gtt = gt.transpose(1, 2, 0)
    p2 = jnp.sum(pred * pred, axis=2).T[:, None, :]
    g2 = jnp.sum(gt * gt, axis=2).T[:, None, :]
    ones = jnp.ones((N, 1, B), f32)
    zeros = jnp.zeros((N, 8 - D - 2, B), f32)
    a_full = jnp.concatenate([pt, p2, ones, zeros], axis=1)
    b_full = jnp.concatenate([-2.0 * gtt, ones, g2, zeros],
                             axis=1)

    nv = (sigma.astype(f32) * sigma.astype(f32)).reshape(1, 1)
    w2 = weights.astype(f32).reshape(B, 1)

    out = pl.pallas_call(
        _vmse_kernel,
        grid=(1,),
        in_specs=[
            pl.BlockSpec(memory_space=pltpu.SMEM),
            pl.BlockSpec((N, 8, B), lambda i: (0, 0, 0)),
            pl.BlockSpec((N, 8, B), lambda i: (0, 0, 0)),
            pl.BlockSpec((B, 1), lambda i: (0, 0)),
        ],
        out_specs=pl.BlockSpec((B, 128), lambda i: (0, 0)),
        out_shape=jax.ShapeDtypeStruct((B, 128), f32),
        scratch_shapes=[pltpu.VMEM((B, B), f32)],
        compiler_params=pltpu.CompilerParams(
            dimension_semantics=("arbitrary",),
            vmem_limit_bytes=48 * 1024 * 1024,
        ),
    )(nv, a_full, b_full, w2)
    return out[:, 0]


# speedup vs baseline: 1.9339x; 1.0594x over previous
import jax
import jax.numpy as jnp
from jax.experimental import pallas as pl
from jax.experimental.pallas import tpu as pltpu

_UNROLL = 32


def _vmse_kernel(nv_ref, a_ref, b_ref, w_ref, o_ref, acc_ref):
    n_pts = a_ref.shape[0]
    bi = a_ref.shape[2]
    b_tot = b_ref.shape[2]

    def dist(n):
        a = a_ref[pl.ds(n, 1)].reshape(8, bi)
        bb = b_ref[pl.ds(n, 1)].reshape(8, b_tot)
        d2 = jax.lax.dot_general(a, bb, (((0,), (0,)), ((), ())),
                                 preferred_element_type=jnp.float32)
        mb = jnp.maximum(d2.astype(jnp.bfloat16),
                         jnp.bfloat16(1e-30))
        return mb * jax.lax.rsqrt(mb)

    acc_ref[:] = jnp.zeros((bi, b_tot), jnp.float32)

    def body(k, _):
        n0 = k * _UNROLL
        s = dist(n0)
        for u in range(1, _UNROLL):
            s = s + dist(n0 + u)
        acc_ref[:] = acc_ref[:] + s.astype(jnp.float32)
        return 0

    jax.lax.fori_loop(0, n_pts // _UNROLL, body, 0)
    acc = acc_ref[:]

    nv = nv_ref[0, 0]
    norms = acc * (1.0 / n_pts)
    logits = (norms * norms) * (-0.5 / nv)
    m = jnp.max(logits, axis=1, keepdims=True)
    ex = jnp.exp(logits - m)
    lse = jnp.log(jnp.sum(ex, axis=1, keepdims=True)) + m
    rows = jax.lax.broadcasted_iota(jnp.int32, (bi, b_tot), 0)
    cols = jax.lax.broadcasted_iota(jnp.int32, (bi, b_tot), 1)
    diag = jnp.sum(jnp.where(rows == cols, logits, 0.0), axis=1,
                   keepdims=True)
    loss = (lse - diag) * (2.0 * nv) * w_ref[:, :1]
    o_ref[:, :] = jnp.broadcast_to(loss, (bi, 128))


@jax.jit
def kernel(pred, gt, weights, sigma):
    B, N, D = pred.shape
    f32 = jnp.float32
    pred = pred.astype(f32)
    gt = gt.astype(f32)

    pt = pred.transpose(1, 2, 0)
    gtt = gt.transpose(1, 2, 0)
    p2 = jnp.sum(pred * pred, axis=2).T[:, None, :]
    g2 = jnp.sum(gt * gt, axis=2).T[:, None, :]
    ones = jnp.ones((N, 1, B), f32)
    zeros = jnp.zeros((N, 8 - D - 2, B), f32)
    a_full = jnp.concatenate([pt, p2, ones, zeros], axis=1)
    b_full = jnp.concatenate([-2.0 * gtt, ones, g2, zeros],
                             axis=1)

    nv = (sigma.astype(f32) * sigma.astype(f32)).reshape(1, 1)
    w2 = weights.astype(f32).reshape(B, 1)

    out = pl.pallas_call(
        _vmse_kernel,
        grid=(1,),
        in_specs=[
            pl.BlockSpec(memory_space=pltpu.SMEM),
            pl.BlockSpec((N, 8, B), lambda i: (0, 0, 0)),
            pl.BlockSpec((N, 8, B), lambda i: (0, 0, 0)),
            pl.BlockSpec((B, 1), lambda i: (0, 0)),
        ],
        out_specs=pl.BlockSpec((B, 128), lambda i: (0, 0)),
        out_shape=jax.ShapeDtypeStruct((B, 128), f32),
        scratch_shapes=[pltpu.VMEM((B, B), f32)],
        compiler_params=pltpu.CompilerParams(
            dimension_semantics=("arbitrary",),
            vmem_limit_bytes=48 * 1024 * 1024,
        ),
    )(nv, a_full, b_full, w2)
    return out[:, 0]


# speedup vs baseline: 1.9640x; 1.0155x over previous
import jax
import jax.numpy as jnp
from jax.experimental import pallas as pl
from jax.experimental.pallas import tpu as pltpu

_UNROLL = 64
_TREE = 32


def _vmse_kernel(nv_ref, a_ref, b_ref, w_ref, o_ref, acc_ref):
    n_pts = a_ref.shape[0]
    bi = a_ref.shape[2]
    b_tot = b_ref.shape[2]

    def dist(n):
        a = a_ref[pl.ds(n, 1)].reshape(8, bi)
        bb = b_ref[pl.ds(n, 1)].reshape(8, b_tot)
        d2 = jax.lax.dot_general(a, bb, (((0,), (0,)), ((), ())),
                                 preferred_element_type=jnp.float32)
        mb = jnp.maximum(d2.astype(jnp.bfloat16),
                         jnp.bfloat16(1e-30))
        return mb * jax.lax.rsqrt(mb)

    acc_ref[:] = jnp.zeros((bi, b_tot), jnp.float32)

    def body(k, _):
        n0 = k * _UNROLL
        parts = []
        for t in range(_UNROLL // _TREE):
            s = dist(n0 + t * _TREE)
            for u in range(1, _TREE):
                s = s + dist(n0 + t * _TREE + u)
            parts.append(s.astype(jnp.float32))
        total = parts[0]
        for p in parts[1:]:
            total = total + p
        acc_ref[:] = acc_ref[:] + total
        return 0

    jax.lax.fori_loop(0, n_pts // _UNROLL, body, 0)
    acc = acc_ref[:]

    nv = nv_ref[0, 0]
    norms = acc * (1.0 / n_pts)
    logits = (norms * norms) * (-0.5 / nv)
    m = jnp.max(logits, axis=1, keepdims=True)
    ex = jnp.exp(logits - m)
    lse = jnp.log(jnp.sum(ex, axis=1, keepdims=True)) + m
    rows = jax.lax.broadcasted_iota(jnp.int32, (bi, b_tot), 0)
    cols = jax.lax.broadcasted_iota(jnp.int32, (bi, b_tot), 1)
    diag = jnp.sum(jnp.where(rows == cols, logits, 0.0), axis=1,
                   keepdims=True)
    loss = (lse - diag) * (2.0 * nv) * w_ref[:, :1]
    o_ref[:, :] = jnp.broadcast_to(loss, (bi, 128))


@jax.jit
def kernel(pred, gt, weights, sigma):
    B, N, D = pred.shape
    f32 = jnp.float32
    pred = pred.astype(f32)
    gt = gt.astype(f32)

    pt = pred.transpose(1, 2, 0)
    gtt = gt.transpose(1, 2, 0)
    p2 = jnp.sum(pred * pred, axis=2).T[:, None, :]
    g2 = jnp.sum(gt * gt, axis=2).T[:, None, :]
    ones = jnp.ones((N, 1, B), f32)
    zeros = jnp.zeros((N, 8 - D - 2, B), f32)
    a_full = jnp.concatenate([pt, p2, ones, zeros], axis=1)
    b_full = jnp.concatenate([-2.0 * gtt, ones, g2, zeros],
                             axis=1)

    nv = (sigma.astype(f32) * sigma.astype(f32)).reshape(1, 1)
    w2 = weights.astype(f32).reshape(B, 1)

    out = pl.pallas_call(
        _vmse_kernel,
        grid=(1,),
        in_specs=[
            pl.BlockSpec(memory_space=pltpu.SMEM),
            pl.BlockSpec((N, 8, B), lambda i: (0, 0, 0)),
            pl.BlockSpec((N, 8, B), lambda i: (0, 0, 0)),
            pl.BlockSpec((B, 1), lambda i: (0, 0)),
        ],
        out_specs=pl.BlockSpec((B, 128), lambda i: (0, 0)),
        out_shape=jax.ShapeDtypeStruct((B, 128), f32),
        scratch_shapes=[pltpu.VMEM((B, B), f32)],
        compiler_params=pltpu.CompilerParams(
            dimension_semantics=("arbitrary",),
            vmem_limit_bytes=48 * 1024 * 1024,
        ),
    )(nv, a_full, b_full, w2)
    return out[:, 0]
